# baseline (speedup 1.0000x reference)
import functools

import jax
import jax.numpy as jnp
from jax import lax
from jax.experimental import pallas as pl
from jax.experimental.pallas import tpu as pltpu
from jax.experimental.pallas import tpu_sc as plsc

BATCH = 16384
FIELDS = 26
EMBED = 64
VAR_LEN = 1000000

FIELDS_PAD = 32
ROW_PAD = 128

NUM_CORES = 2
NUM_SUBCORES = 16
NUM_WORKERS = NUM_CORES * NUM_SUBCORES

B_PER_W = BATCH // NUM_WORKERS
CHUNK = 128
HALVES = B_PER_W // CHUNK
N_STEPS = FIELDS * HALVES
NBUF = 4


def _gather_body(idx_hbm, table_hbm, out_hbm, idx_v, rows_v, g_sems, w_sems):
    wid = lax.axis_index("s") * NUM_CORES + lax.axis_index("c")
    b0 = wid * B_PER_W

    def gather(j, b):
        f = j // HALVES
        h = j % HALVES
        pltpu.sync_copy(
            idx_hbm.at[pl.ds(f * BATCH + b0 + h * CHUNK, CHUNK)], idx_v.at[b]
        )
        pltpu.async_copy(table_hbm.at[idx_v.at[b]], rows_v.at[b], g_sems.at[b])

    def wait_gather(b):
        pltpu.make_async_copy(
            table_hbm.at[idx_v.at[b]], rows_v.at[b], g_sems.at[b]
        ).wait()

    def writeback(j, b):
        f = j // HALVES
        h = j % HALVES
        pltpu.async_copy(
            rows_v.at[b, pl.ds(0, CHUNK), pl.ds(0, EMBED)],
            out_hbm.at[pl.ds(b0 + h * CHUNK, CHUNK), f, pl.ds(0, EMBED)],
            w_sems.at[b],
        )

    def wait_writeback(b):
        pltpu.make_async_copy(
            rows_v.at[b, pl.ds(0, CHUNK), pl.ds(0, EMBED)],
            out_hbm.at[pl.ds(b0, CHUNK), 0, pl.ds(0, EMBED)],
            w_sems.at[b],
        ).wait()

    for b in range(NBUF):
        gather(b, b)

    @pl.loop(0, N_STEPS, step=NBUF)
    def _outer(j0):
        for b in range(NBUF):
            j = j0 + b
            wait_gather(b)
            writeback(j, b)

            @pl.when(j + NBUF < N_STEPS)
            def _refill():
                wait_writeback(b)
                gather(j + NBUF, b)

    for b in range(NBUF):
        wait_writeback(b)


TP_BC = 32768
TP_NBLK = -(-VAR_LEN // TP_BC)


def _tp_body(x_ref, o_ref):
    o_ref[:, :EMBED] = x_ref[...].T


def _pad_transpose(wT):
    return pl.pallas_call(
        _tp_body,
        grid=(TP_NBLK,),
        in_specs=[pl.BlockSpec((EMBED, TP_BC), lambda j: (0, j))],
        out_specs=pl.BlockSpec((TP_BC, ROW_PAD), lambda j: (j, 0)),
        out_shape=jax.ShapeDtypeStruct((VAR_LEN, ROW_PAD), jnp.float32),
    )(wT)


@jax.jit
def _embed(idx_flat, wT):
    table_pad = _pad_transpose(wT)
    mesh = plsc.VectorSubcoreMesh(core_axis_name="c", subcore_axis_name="s")
    k = functools.partial(
        pl.kernel,
        out_type=jax.ShapeDtypeStruct((BATCH, FIELDS_PAD, ROW_PAD), jnp.float32),
        mesh=mesh,
        scratch_types=[
            pltpu.VMEM((NBUF, CHUNK), jnp.int32),
            pltpu.VMEM((NBUF, CHUNK, ROW_PAD), jnp.float32),
            pltpu.SemaphoreType.DMA((NBUF,)),
            pltpu.SemaphoreType.DMA((NBUF,)),
        ],
        compiler_params=pltpu.CompilerParams(use_tc_tiling_on_sc=False),
    )(_gather_body)
    return k(idx_flat, table_pad)


def kernel(input, weight):
    idx_flat = input.astype(jnp.int32).T.reshape(BATCH * FIELDS)
    out_pad = _embed(idx_flat, weight.T)
    return out_pad[:, :FIELDS, :EMBED]


# speedup vs baseline: 1.0488x; 1.0488x over previous
import functools

import jax
import jax.numpy as jnp
from jax import lax
from jax.experimental import pallas as pl
from jax.experimental.pallas import tpu as pltpu
from jax.experimental.pallas import tpu_sc as plsc

BATCH = 16384
FIELDS = 26
EMBED = 64
VAR_LEN = 1000000

FIELDS_PAD = 32
ROW_PAD = 128

NUM_CORES = 2
NUM_SUBCORES = 16
NUM_WORKERS = NUM_CORES * NUM_SUBCORES

B_PER_W = BATCH // NUM_WORKERS
CHUNK = 256
HALVES = B_PER_W // CHUNK
N_STEPS = FIELDS * HALVES
NBUF = 2


def _gather_body(idx_hbm, table_hbm, out_hbm, idx_v, rows_v, g_sems, w_sems):
    wid = lax.axis_index("s") * NUM_CORES + lax.axis_index("c")
    b0 = wid * B_PER_W

    pltpu.sync_copy(
        idx_hbm.at[pl.ds(0, FIELDS), pl.ds(b0, B_PER_W)], idx_v
    )

    def idx_chunk(j):
        f = j // HALVES
        h = j % HALVES
        return idx_v.at[f, pl.ds(h * CHUNK, CHUNK)]

    def gather(j, b):
        pltpu.async_copy(table_hbm.at[idx_chunk(j)], rows_v.at[b], g_sems.at[b])

    def wait_gather(b):
        pltpu.make_async_copy(
            table_hbm.at[idx_chunk(0)], rows_v.at[b], g_sems.at[b]
        ).wait()

    def writeback(j, b):
        f = j // HALVES
        h = j % HALVES
        pltpu.async_copy(
            rows_v.at[b, pl.ds(0, CHUNK), pl.ds(0, EMBED)],
            out_hbm.at[pl.ds(b0 + h * CHUNK, CHUNK), f, pl.ds(0, EMBED)],
            w_sems.at[b],
        )

    def wait_writeback(b):
        pltpu.make_async_copy(
            rows_v.at[b, pl.ds(0, CHUNK), pl.ds(0, EMBED)],
            out_hbm.at[pl.ds(b0, CHUNK), 0, pl.ds(0, EMBED)],
            w_sems.at[b],
        ).wait()

    for b in range(NBUF):
        gather(b, b)

    @pl.loop(0, N_STEPS, step=NBUF)
    def _outer(j0):
        for b in range(NBUF):
            j = j0 + b
            wait_gather(b)
            writeback(j, b)

            @pl.when(j + NBUF < N_STEPS)
            def _refill():
                wait_writeback(b)
                gather(j + NBUF, b)

    for b in range(NBUF):
        wait_writeback(b)


TP_BC = 32768
TP_NBLK = -(-VAR_LEN // TP_BC)


def _tp_body(x_ref, o_ref):
    o_ref[:, :EMBED] = x_ref[...].T


def _pad_transpose(wT):
    return pl.pallas_call(
        _tp_body,
        grid=(TP_NBLK,),
        in_specs=[pl.BlockSpec((EMBED, TP_BC), lambda j: (0, j))],
        out_specs=pl.BlockSpec((TP_BC, ROW_PAD), lambda j: (j, 0)),
        out_shape=jax.ShapeDtypeStruct((VAR_LEN, ROW_PAD), jnp.float32),
    )(wT)


@jax.jit
def _embed(idx2, wT):
    table_pad = _pad_transpose(wT)
    mesh = plsc.VectorSubcoreMesh(core_axis_name="c", subcore_axis_name="s")
    k = functools.partial(
        pl.kernel,
        out_type=jax.ShapeDtypeStruct((BATCH, FIELDS_PAD, ROW_PAD), jnp.float32),
        mesh=mesh,
        scratch_types=[
            pltpu.VMEM((FIELDS, B_PER_W), jnp.int32),
            pltpu.VMEM((NBUF, CHUNK, ROW_PAD), jnp.float32),
            pltpu.SemaphoreType.DMA((NBUF,)),
            pltpu.SemaphoreType.DMA((NBUF,)),
        ],
        compiler_params=pltpu.CompilerParams(use_tc_tiling_on_sc=False),
    )(_gather_body)
    return k(idx2, table_pad)


def kernel(input, weight):
    idx2 = input.astype(jnp.int32).T
    out_pad = _embed(idx2, weight.T)
    return out_pad[:, :FIELDS, :EMBED]


# speedup vs baseline: 1.0506x; 1.0016x over previous
import functools

import jax
import jax.numpy as jnp
from jax import lax
from jax.experimental import pallas as pl
from jax.experimental.pallas import tpu as pltpu
from jax.experimental.pallas import tpu_sc as plsc

BATCH = 16384
FIELDS = 26
EMBED = 64
VAR_LEN = 1000000

FIELDS_PAD = 32
ROW_PAD = 128

NUM_CORES = 2
NUM_SUBCORES = 16
NUM_WORKERS = NUM_CORES * NUM_SUBCORES

B_PER_W = BATCH // NUM_WORKERS
CHUNK = 128
HALVES = B_PER_W // CHUNK
N_STEPS = FIELDS * HALVES
NBUF = 4


def _gather_body(idx_hbm, table_hbm, out_hbm, idx_v, rows_v, g_sems, w_sems):
    wid = lax.axis_index("s") * NUM_CORES + lax.axis_index("c")
    b0 = wid * B_PER_W

    pltpu.sync_copy(
        idx_hbm.at[pl.ds(0, FIELDS), pl.ds(b0, B_PER_W)], idx_v
    )

    def idx_chunk(j):
        f = j // HALVES
        h = j % HALVES
        return idx_v.at[f, pl.ds(h * CHUNK, CHUNK)]

    def gather(j, b):
        pltpu.async_copy(table_hbm.at[idx_chunk(j)], rows_v.at[b], g_sems.at[b])

    def wait_gather(b):
        pltpu.make_async_copy(
            table_hbm.at[idx_chunk(0)], rows_v.at[b], g_sems.at[b]
        ).wait()

    def writeback(j, b):
        f = j // HALVES
        h = j % HALVES
        pltpu.async_copy(
            rows_v.at[b, pl.ds(0, CHUNK), pl.ds(0, EMBED)],
            out_hbm.at[pl.ds(b0 + h * CHUNK, CHUNK), f, pl.ds(0, EMBED)],
            w_sems.at[b],
        )

    def wait_writeback(b):
        pltpu.make_async_copy(
            rows_v.at[b, pl.ds(0, CHUNK), pl.ds(0, EMBED)],
            out_hbm.at[pl.ds(b0, CHUNK), 0, pl.ds(0, EMBED)],
            w_sems.at[b],
        ).wait()

    for b in range(NBUF):
        gather(b, b)

    @pl.loop(0, N_STEPS, step=NBUF)
    def _outer(j0):
        for b in range(NBUF):
            j = j0 + b
            wait_gather(b)
            writeback(j, b)

            @pl.when(j + NBUF < N_STEPS)
            def _refill():
                wait_writeback(b)
                gather(j + NBUF, b)

    for b in range(NBUF):
        wait_writeback(b)


TP_BC = 32768
TP_NBLK = -(-VAR_LEN // TP_BC)


def _tp_body(x_ref, o_ref):
    o_ref[:, :EMBED] = x_ref[...].T


def _pad_transpose(wT):
    return pl.pallas_call(
        _tp_body,
        grid=(TP_NBLK,),
        in_specs=[pl.BlockSpec((EMBED, TP_BC), lambda j: (0, j))],
        out_specs=pl.BlockSpec((TP_BC, ROW_PAD), lambda j: (j, 0)),
        out_shape=jax.ShapeDtypeStruct((VAR_LEN, ROW_PAD), jnp.float32),
    )(wT)


@jax.jit
def _embed(idx2, wT):
    table_pad = _pad_transpose(wT)
    mesh = plsc.VectorSubcoreMesh(core_axis_name="c", subcore_axis_name="s")
    k = functools.partial(
        pl.kernel,
        out_type=jax.ShapeDtypeStruct((BATCH, FIELDS_PAD, ROW_PAD), jnp.float32),
        mesh=mesh,
        scratch_types=[
            pltpu.VMEM((FIELDS, B_PER_W), jnp.int32),
            pltpu.VMEM((NBUF, CHUNK, ROW_PAD), jnp.float32),
            pltpu.SemaphoreType.DMA((NBUF,)),
            pltpu.SemaphoreType.DMA((NBUF,)),
        ],
        compiler_params=pltpu.CompilerParams(use_tc_tiling_on_sc=False),
    )(_gather_body)
    return k(idx2, table_pad)


def kernel(input, weight):
    idx2 = input.astype(jnp.int32).T
    out_pad = _embed(idx2, weight.T)
    return out_pad[:, :FIELDS, :EMBED]
